# baseline (speedup 1.0000x reference)
import functools

import jax
import jax.numpy as jnp
from jax import lax
from jax.experimental import pallas as pl
from jax.experimental.pallas import tpu as pltpu
from jax.experimental.pallas import tpu_sc as plsc

B = 16384
D = 64
NC = 1
NS = 16
NW = NC * NS
B_PER_W = B // NW
CHUNK = 64
NCHUNK = B_PER_W // CHUNK
L = 16


def _make_gather():
    mesh = plsc.VectorSubcoreMesh(core_axis_name="c", subcore_axis_name="s", num_cores=1)

    @functools.partial(
        pl.kernel,
        mesh=mesh,
        out_type=jax.ShapeDtypeStruct((B, D), jnp.float32),
        scratch_types=[
            pltpu.VMEM((B_PER_W,), jnp.int32),
            pltpu.VMEM((NCHUNK, CHUNK), jnp.int32),
            pltpu.VMEM((2, CHUNK, 2 * D), jnp.float32),
            pltpu.VMEM((2, CHUNK, D), jnp.float32),
            pltpu.SemaphoreType.DMA,
            pltpu.SemaphoreType.DMA,
            pltpu.SemaphoreType.DMA,
            pltpu.SemaphoreType.DMA,
        ],
    )
    def gather(table_hbm, idx_hbm, out_hbm, idx_v, pid_v, slab_v, row_v,
               sg0, sg1, so0, so1):
        wid = lax.axis_index("s") * NC + lax.axis_index("c")
        base = wid * B_PER_W
        pltpu.sync_copy(idx_hbm.at[pl.ds(base, B_PER_W)], idx_v)
        for k in range(B_PER_W // L):
            v = idx_v[pl.ds(k * L, L)] >> 1
            pid_v[k // (CHUNK // L), pl.ds((k % (CHUNK // L)) * L, L)] = v

        gsems = [sg0, sg1]
        osems = [so0, so1]

        def fire_gather(q):
            b = q % 2
            return pltpu.async_copy(
                table_hbm.at[pid_v.at[q]], slab_v.at[b], gsems[b])

        def extract(q):
            b = q % 2

            def body(g, _):
                v = idx_v[pl.ds(q * CHUNK + g * L, L)]
                for l in range(L):
                    i = g * L + l
                    j = (v[l] & 1) * D
                    for c in range(D // L):
                        row_v[b, i, pl.ds(c * L, L)] = (
                            slab_v[b, i, pl.ds(j + c * L, L)])
                return 0

            lax.fori_loop(0, CHUNK // L, body, 0)

        def fire_out(q):
            b = q % 2
            return pltpu.async_copy(
                row_v.at[b], out_hbm.at[pl.ds(base + q * CHUNK, CHUNK)],
                osems[b])

        g_copies = {0: fire_gather(0)}
        o_copies = {}
        for q in range(NCHUNK):
            if q + 1 < NCHUNK:
                g_copies[q + 1] = fire_gather(q + 1)
            g_copies.pop(q).wait()
            if q - 2 in o_copies:
                o_copies.pop(q - 2).wait()
            extract(q)
            o_copies[q] = fire_out(q)
        for q in sorted(o_copies):
            o_copies.pop(q).wait()

    return gather


_gather = _make_gather()


def kernel(class_ids, table):
    table2 = table.reshape(500000, 2 * D)
    out = _gather(table2, class_ids)
    return out.reshape(B, 1, D)


# speedup vs baseline: 1.0503x; 1.0503x over previous
import functools

import jax
import jax.numpy as jnp
from jax import lax
from jax.experimental import pallas as pl
from jax.experimental.pallas import tpu as pltpu
from jax.experimental.pallas import tpu_sc as plsc

B = 16384
D = 64
NC = 2
NS = 16
NW = NC * NS
B_PER_W = B // NW


def _make_gather():
    mesh = plsc.VectorSubcoreMesh(core_axis_name="c", subcore_axis_name="s")

    @functools.partial(
        pl.kernel,
        mesh=mesh,
        out_type=jax.ShapeDtypeStruct((B, D), jnp.float32),
        scratch_types=[
            pltpu.VMEM((B_PER_W,), jnp.int32),
            pltpu.SemaphoreType.DMA,
        ],
    )
    def gather(table_hbm, idx_hbm, out_hbm, idx_v, sem):
        wid = lax.axis_index("s") * NC + lax.axis_index("c")
        base = wid * B_PER_W
        pltpu.sync_copy(idx_hbm.at[pl.ds(base, B_PER_W)], idx_v)

        def fire(g, _):
            v = idx_v[pl.ds(g * 16, 16)]
            for l in range(16):
                pltpu.async_copy(
                    table_hbm.at[pl.ds(v[l], 1), :],
                    out_hbm.at[pl.ds(base + g * 16 + l, 1), :],
                    sem,
                )
            return 0

        lax.fori_loop(0, B_PER_W // 16, fire, 0)

        def drain(i, _):
            pltpu.make_async_copy(
                table_hbm.at[pl.ds(0, 1), :],
                out_hbm.at[pl.ds(base, 1), :],
                sem,
            ).wait()
            return 0

        lax.fori_loop(0, B_PER_W, drain, 0)

    return gather


_gather = _make_gather()


def kernel(class_ids, table):
    out = _gather(table, class_ids)
    return out.reshape(B, 1, D)
